# Initial kernel scaffold; baseline (speedup 1.0000x reference)
import jax
import jax.numpy as jnp
from jax.experimental import pallas as pl


def kernel(x, weight, bias, popup_score):
    raise NotImplementedError("write your pallas kernel here")


# speedup vs baseline: 59.7565x; 59.7565x over previous
import functools

import jax
import jax.numpy as jnp
from jax import lax
from jax.experimental import pallas as pl
from jax.experimental.pallas import tpu as pltpu
from jax.experimental.pallas import tpu_sc as plsc

N_TOT = 8192 * 1024
RANK_J = N_TOT // 2
NW = 32
PER_W = N_TOT // NW
CHUNK = 8192
NCHUNK = PER_W // CHUNK
NBINS = 2048
L = 16

@functools.cache
def _mesh():
    return plsc.VectorSubcoreMesh(core_axis_name="c", subcore_axis_name="s",
                                  num_cores=2, num_subcores=16)


def _worker_id():
    return lax.axis_index("s") * 2 + lax.axis_index("c")


def _find_bin(tot_ref, target):
    zeros = jnp.zeros((L,), jnp.int32)
    ones = jnp.ones((L,), jnp.int32)

    def body(c, carry):
        csum, bcnt, below = carry
        v = tot_ref[pl.ds(c * L, L)]
        cs = plsc.cumsum(v) + csum
        flags = cs < target
        bcnt = bcnt + jnp.where(flags, ones, zeros)
        below = below + jnp.where(flags, v, zeros)
        csum = csum + jnp.sum(v)
        return csum, bcnt, below

    _, bcnt, below = lax.fori_loop(0, NBINS // L, body, (zeros, zeros, zeros))
    return jnp.sum(bcnt), jnp.sum(below)


def _reduce_hist(h_ref, staging_ref, tot_ref):
    for b in range(4):
        pltpu.sync_copy(h_ref.at[pl.ds(b * 8 * NBINS, 8 * NBINS)], staging_ref)

        def body(c, _, _b=b):
            acc = staging_ref[pl.ds(c * L, L)]
            for r in range(1, 8):
                acc = acc + staging_ref[pl.ds(r * NBINS + c * L, L)]
            if _b == 0:
                tot_ref[pl.ds(c * L, L)] = acc
            else:
                tot_ref[pl.ds(c * L, L)] = tot_ref[pl.ds(c * L, L)] + acc
            return 0

        lax.fori_loop(0, NBINS // L, body, 0)


def _consume(buf_ref, hist_ref, laneoff, ones, mode, sel):
    def body(i, _):
        for u in range(4):
            v = buf_ref[pl.ds((i * 4 + u) * L, L)]
            ab = v & jnp.int32(0x7FFFFFFF)
            if mode == 1:
                plsc.addupdate_scatter(hist_ref, [laneoff + (ab >> 20)], ones)
            elif mode == 2:
                pred = (ab >> 20) == sel
                b = (ab >> 9) & jnp.int32(0x7FF)
                plsc.addupdate_scatter(hist_ref, [laneoff + b], ones,
                                       mask=pred)
            else:
                pred = (ab >> 9) == sel
                b = ab & jnp.int32(0x1FF)
                plsc.addupdate_scatter(hist_ref, [laneoff + b], ones,
                                       mask=pred)
        return 0

    lax.fori_loop(0, CHUNK // (4 * L), body, 0)


def _hist_body(ps_ref, out_ref, buf0, buf1, hist, totrow, staging, tot,
               sem0, sem1, mode, h1=None, h2=None):
    w = _worker_id()
    laneoff = lax.iota(jnp.int32, L) * NBINS
    ones = jnp.ones((L,), jnp.int32)

    if mode == 1:
        sel = None
    elif mode == 2:
        _reduce_hist(h1, staging, tot)
        b1, _ = _find_bin(tot, jnp.int32(RANK_J + 1))
        sel = b1
    else:
        _reduce_hist(h1, staging, tot)
        b1, below1 = _find_bin(tot, jnp.int32(RANK_J + 1))
        _reduce_hist(h2, staging, tot)
        b2, _ = _find_bin(tot, jnp.int32(RANK_J + 1) - below1)
        sel = (b1 << 11) | b2

    zeros = jnp.zeros((L,), jnp.int32)

    def zbody(c, _):
        for r in range(L):
            hist[pl.ds(r * NBINS + c * L, L)] = zeros
        return 0

    lax.fori_loop(0, NBINS // L, zbody, 0)

    base = w * PER_W

    def src(c):
        return ps_ref.at[pl.ds(base + c * CHUNK, CHUNK)]

    cp0 = pltpu.async_copy(src(0), buf0, sem0)
    cp1 = pltpu.async_copy(src(1), buf1, sem1)

    def pair(p, _):
        c = p * 2
        pltpu.make_async_copy(src(c), buf0, sem0).wait()
        _consume(buf0, hist, laneoff, ones, mode, sel)

        @pl.when(c + 2 < NCHUNK)
        def _():
            pltpu.async_copy(src(c + 2), buf0, sem0)

        pltpu.make_async_copy(src(c + 1), buf1, sem1).wait()
        _consume(buf1, hist, laneoff, ones, mode, sel)

        @pl.when(c + 3 < NCHUNK)
        def _():
            pltpu.async_copy(src(c + 3), buf1, sem1)

        return 0

    lax.fori_loop(0, NCHUNK // 2, pair, 0)
    del cp0, cp1

    def rbody(c, _):
        acc = hist[pl.ds(c * L, L)]
        for r in range(1, L):
            acc = acc + hist[pl.ds(r * NBINS + c * L, L)]
        totrow[pl.ds(c * L, L)] = acc
        return 0

    lax.fori_loop(0, NBINS // L, rbody, 0)
    pltpu.sync_copy(totrow, out_ref.at[pl.ds(w * NBINS, NBINS)])


@functools.cache
def _make_sc_pass(mode):
    scratch = [
        pltpu.VMEM((CHUNK,), jnp.int32),
        pltpu.VMEM((CHUNK,), jnp.int32),
        pltpu.VMEM((L * NBINS,), jnp.int32),
        pltpu.VMEM((NBINS,), jnp.int32),
        pltpu.VMEM((8 * NBINS,), jnp.int32),
        pltpu.VMEM((NBINS,), jnp.int32),
        pltpu.SemaphoreType.DMA,
        pltpu.SemaphoreType.DMA,
    ]
    out = jax.ShapeDtypeStruct((NW * NBINS,), jnp.int32)
    cp = pltpu.CompilerParams(needs_layout_passes=False)

    if mode == 1:
        @functools.partial(pl.kernel, out_type=out, mesh=_mesh(),
                           scratch_types=scratch, compiler_params=cp)
        def k(ps, o, b0, b1, hist, totrow, staging, tot, s0, s1):
            _hist_body(ps, o, b0, b1, hist, totrow, staging, tot, s0, s1, 1)
        return k
    if mode == 2:
        @functools.partial(pl.kernel, out_type=out, mesh=_mesh(),
                           scratch_types=scratch, compiler_params=cp)
        def k(ps, h1, o, b0, b1, hist, totrow, staging, tot, s0, s1):
            _hist_body(ps, o, b0, b1, hist, totrow, staging, tot, s0, s1, 2,
                       h1=h1)
        return k

    @functools.partial(pl.kernel, out_type=out, mesh=_mesh(),
                       scratch_types=scratch, compiler_params=cp)
    def k(ps, h1, h2, o, b0, b1, hist, totrow, staging, tot, s0, s1):
        _hist_body(ps, o, b0, b1, hist, totrow, staging, tot, s0, s1, 3,
                   h1=h1, h2=h2)
    return k


@functools.cache
def _make_sc_finalize():
    @functools.partial(pl.kernel,
                       out_type=jax.ShapeDtypeStruct((L,), jnp.float32),
                       mesh=_mesh(),
                       scratch_types=[
                           pltpu.VMEM((8 * NBINS,), jnp.int32),
                           pltpu.VMEM((NBINS,), jnp.int32),
                           pltpu.VMEM((L,), jnp.float32),
                       ],
                       compiler_params=pltpu.CompilerParams(
                           needs_layout_passes=False))
    def _sc_finalize(h1, h2, h3, out, staging, tot, tvec):
        w = _worker_id()

        @pl.when(w == 0)
        def _():
            _reduce_hist(h1, staging, tot)
            b1, below1 = _find_bin(tot, jnp.int32(RANK_J + 1))
            t2 = jnp.int32(RANK_J + 1) - below1
            _reduce_hist(h2, staging, tot)
            b2, below2 = _find_bin(tot, t2)
            t3 = t2 - below2
            _reduce_hist(h3, staging, tot)
            b3, _ = _find_bin(tot, t3)
            tbits = (b1 << 20) | (b2 << 9) | b3
            tvec[...] = plsc.bitcast(jnp.broadcast_to(tbits, (L,)),
                                     jnp.float32)
            pltpu.sync_copy(tvec, out)

    return _sc_finalize


def _tc_prep_body(x_ref, w_ref, ps_ref, t_ref, x16_ref, wm_ref):
    x16_ref[...] = x_ref[...].astype(jnp.bfloat16)
    t = t_ref[0, 0]
    m = jnp.abs(ps_ref[...]) >= t
    wm_ref[...] = jnp.where(m, w_ref[...], 0.0).astype(jnp.bfloat16)


def _tc_prep(x, weight, ps_top, t11):
    grid = 32
    return pl.pallas_call(
        _tc_prep_body,
        grid=(grid,),
        in_specs=[
            pl.BlockSpec((8192 // grid, 2048), lambda g: (g, 0)),
            pl.BlockSpec((2048 // grid, 2048), lambda g: (g, 0)),
            pl.BlockSpec((2048 // grid, 2048), lambda g: (g, 0)),
            pl.BlockSpec(memory_space=pltpu.SMEM),
        ],
        out_specs=[
            pl.BlockSpec((8192 // grid, 2048), lambda g: (g, 0)),
            pl.BlockSpec((2048 // grid, 2048), lambda g: (g, 0)),
        ],
        out_shape=[
            jax.ShapeDtypeStruct((8192, 2048), jnp.bfloat16),
            jax.ShapeDtypeStruct((2048, 2048), jnp.bfloat16),
        ],
    )(x, weight, ps_top, t11)


def _tc_matmul_body(x_ref, w_ref, b_ref, o_ref):
    acc = lax.dot_general(
        x_ref[...], w_ref[...], (((1,), (1,)), ((), ())),
        preferred_element_type=jnp.float32)
    o_ref[...] = acc + b_ref[...]


def _tc_matmul(x16, wm, bias2d):
    bm, bn = 512, 512
    return pl.pallas_call(
        _tc_matmul_body,
        grid=(2048 // bn, 8192 // bm),
        in_specs=[
            pl.BlockSpec((bm, 2048), lambda j, i: (i, 0)),
            pl.BlockSpec((bn, 2048), lambda j, i: (j, 0)),
            pl.BlockSpec((1, bn), lambda j, i: (0, j)),
        ],
        out_specs=pl.BlockSpec((bm, bn), lambda j, i: (i, j)),
        out_shape=jax.ShapeDtypeStruct((8192, 2048), jnp.float32),
        compiler_params=pltpu.CompilerParams(
            dimension_semantics=("arbitrary", "arbitrary")),
    )(x16, wm, bias2d)


def kernel(x, weight, bias, popup_score):
    ps_bits = lax.bitcast_convert_type(popup_score, jnp.int32).reshape(-1)
    h1 = _make_sc_pass(1)(ps_bits)
    h2 = _make_sc_pass(2)(ps_bits, h1)
    h3 = _make_sc_pass(3)(ps_bits, h1, h2)
    t = _make_sc_finalize()(h1, h2, h3)
    t11 = t[:1].reshape(1, 1)
    x16, wm = _tc_prep(x, weight, popup_score[:2048], t11)
    return _tc_matmul(x16, wm, bias.reshape(1, 2048))


# speedup vs baseline: 71.5163x; 1.1968x over previous
import functools

import jax
import jax.numpy as jnp
from jax import lax
from jax.experimental import pallas as pl
from jax.experimental.pallas import tpu as pltpu
from jax.experimental.pallas import tpu_sc as plsc

N_TOT = 8192 * 1024
RANK_J = N_TOT // 2
NW = 32
PER_W = N_TOT // NW
CHUNK = 8192
NCHUNK = PER_W // CHUNK
NBINS = 2048
L = 16

@functools.cache
def _mesh():
    return plsc.VectorSubcoreMesh(core_axis_name="c", subcore_axis_name="s",
                                  num_cores=2, num_subcores=16)


def _worker_id():
    return lax.axis_index("s") * 2 + lax.axis_index("c")


def _find_bin(tot_ref, target):
    zeros = jnp.zeros((L,), jnp.int32)
    ones = jnp.ones((L,), jnp.int32)

    def body(c, carry):
        csum, bcnt, below = carry
        v = tot_ref[pl.ds(c * L, L)]
        cs = plsc.cumsum(v) + csum
        flags = cs < target
        bcnt = bcnt + jnp.where(flags, ones, zeros)
        below = below + jnp.where(flags, v, zeros)
        csum = csum + jnp.sum(v)
        return csum, bcnt, below

    _, bcnt, below = lax.fori_loop(0, NBINS // L, body, (zeros, zeros, zeros))
    return jnp.sum(bcnt), jnp.sum(below)


def _reduce_hist(h_ref, staging_ref, tot_ref):
    pltpu.sync_copy(h_ref, staging_ref)

    def body(c, _):
        acc = staging_ref[pl.ds(c * L, L)]
        for r in range(1, NW):
            acc = acc + staging_ref[pl.ds(r * NBINS + c * L, L)]
        tot_ref[pl.ds(c * L, L)] = acc
        return 0

    lax.fori_loop(0, NBINS // L, body, 0)


def _consume(buf_ref, hist_ref, laneoff, ones, mode, sel):
    UNROLL = 16

    def body(i, _):
        for u in range(UNROLL):
            v = plsc.bitcast(buf_ref[pl.ds((i * UNROLL + u) * L, L)],
                             jnp.int32)
            ab = v & jnp.int32(0x7FFFFFFF)
            if mode == 1:
                plsc.addupdate_scatter(hist_ref, [laneoff + (ab >> 20)], ones)
            elif mode == 2:
                pred = (ab >> 20) == sel
                b = (ab >> 9) & jnp.int32(0x7FF)
                plsc.addupdate_scatter(hist_ref, [laneoff + b], ones,
                                       mask=pred)
            else:
                pred = (ab >> 9) == sel
                b = ab & jnp.int32(0x1FF)
                plsc.addupdate_scatter(hist_ref, [laneoff + b], ones,
                                       mask=pred)
        return 0

    lax.fori_loop(0, CHUNK // (UNROLL * L), body, 0)


def _hist_body(ps_ref, out_ref, buf0, buf1, hist, totrow, staging, tot,
               sem0, sem1, mode, h1=None, h2=None):
    w = _worker_id()
    laneoff = lax.iota(jnp.int32, L) * NBINS
    ones = jnp.ones((L,), jnp.int32)

    if mode == 1:
        sel = None
    elif mode == 2:
        _reduce_hist(h1, staging, tot)
        b1, _ = _find_bin(tot, jnp.int32(RANK_J + 1))
        sel = b1
    else:
        _reduce_hist(h1, staging, tot)
        b1, below1 = _find_bin(tot, jnp.int32(RANK_J + 1))
        _reduce_hist(h2, staging, tot)
        b2, _ = _find_bin(tot, jnp.int32(RANK_J + 1) - below1)
        sel = (b1 << 11) | b2

    zeros = jnp.zeros((L,), jnp.int32)

    def zbody(c, _):
        for r in range(L):
            hist[pl.ds(r * NBINS + c * L, L)] = zeros
        return 0

    lax.fori_loop(0, NBINS // L, zbody, 0)

    base = w * PER_W

    def src(c):
        return ps_ref.at[pl.ds(base + c * CHUNK, CHUNK)]

    cp0 = pltpu.async_copy(src(0), buf0, sem0)
    cp1 = pltpu.async_copy(src(1), buf1, sem1)

    def pair(p, _):
        c = p * 2
        pltpu.make_async_copy(src(c), buf0, sem0).wait()
        _consume(buf0, hist, laneoff, ones, mode, sel)

        @pl.when(c + 2 < NCHUNK)
        def _():
            pltpu.async_copy(src(c + 2), buf0, sem0)

        pltpu.make_async_copy(src(c + 1), buf1, sem1).wait()
        _consume(buf1, hist, laneoff, ones, mode, sel)

        @pl.when(c + 3 < NCHUNK)
        def _():
            pltpu.async_copy(src(c + 3), buf1, sem1)

        return 0

    lax.fori_loop(0, NCHUNK // 2, pair, 0)
    del cp0, cp1

    def rbody(c, _):
        acc = hist[pl.ds(c * L, L)]
        for r in range(1, L):
            acc = acc + hist[pl.ds(r * NBINS + c * L, L)]
        totrow[pl.ds(c * L, L)] = acc
        return 0

    lax.fori_loop(0, NBINS // L, rbody, 0)
    pltpu.sync_copy(totrow, out_ref.at[pl.ds(w * NBINS, NBINS)])


@functools.cache
def _make_sc_pass(mode):
    scratch = [
        pltpu.VMEM((CHUNK,), jnp.float32),
        pltpu.VMEM((CHUNK,), jnp.float32),
        pltpu.VMEM((L * NBINS,), jnp.int32),
        pltpu.VMEM((NBINS,), jnp.int32),
        pltpu.VMEM((NW * NBINS,), jnp.int32),
        pltpu.VMEM((NBINS,), jnp.int32),
        pltpu.SemaphoreType.DMA,
        pltpu.SemaphoreType.DMA,
    ]
    out = jax.ShapeDtypeStruct((NW * NBINS,), jnp.int32)
    cp = pltpu.CompilerParams(needs_layout_passes=False)

    if mode == 1:
        @functools.partial(pl.kernel, out_type=out, mesh=_mesh(),
                           scratch_types=scratch, compiler_params=cp)
        def k(ps, o, b0, b1, hist, totrow, staging, tot, s0, s1):
            _hist_body(ps, o, b0, b1, hist, totrow, staging, tot, s0, s1, 1)
        return k
    if mode == 2:
        @functools.partial(pl.kernel, out_type=out, mesh=_mesh(),
                           scratch_types=scratch, compiler_params=cp)
        def k(ps, h1, o, b0, b1, hist, totrow, staging, tot, s0, s1):
            _hist_body(ps, o, b0, b1, hist, totrow, staging, tot, s0, s1, 2,
                       h1=h1)
        return k

    @functools.partial(pl.kernel, out_type=out, mesh=_mesh(),
                       scratch_types=scratch, compiler_params=cp)
    def k(ps, h1, h2, o, b0, b1, hist, totrow, staging, tot, s0, s1):
        _hist_body(ps, o, b0, b1, hist, totrow, staging, tot, s0, s1, 3,
                   h1=h1, h2=h2)
    return k


@functools.cache
def _make_sc_finalize():
    @functools.partial(pl.kernel,
                       out_type=jax.ShapeDtypeStruct((L,), jnp.float32),
                       mesh=_mesh(),
                       scratch_types=[
                           pltpu.VMEM((NW * NBINS,), jnp.int32),
                           pltpu.VMEM((NBINS,), jnp.int32),
                           pltpu.VMEM((L,), jnp.float32),
                       ],
                       compiler_params=pltpu.CompilerParams(
                           needs_layout_passes=False))
    def _sc_finalize(h1, h2, h3, out, staging, tot, tvec):
        w = _worker_id()

        @pl.when(w == 0)
        def _():
            _reduce_hist(h1, staging, tot)
            b1, below1 = _find_bin(tot, jnp.int32(RANK_J + 1))
            t2 = jnp.int32(RANK_J + 1) - below1
            _reduce_hist(h2, staging, tot)
            b2, below2 = _find_bin(tot, t2)
            t3 = t2 - below2
            _reduce_hist(h3, staging, tot)
            b3, _ = _find_bin(tot, t3)
            tbits = (b1 << 20) | (b2 << 9) | b3
            tvec[...] = plsc.bitcast(jnp.broadcast_to(tbits, (L,)),
                                     jnp.float32)
            pltpu.sync_copy(tvec, out)

    return _sc_finalize


def _tc_prep_body(w_ref, ps_ref, t_ref, wm_ref):
    t = t_ref[0, 0]
    m = jnp.abs(ps_ref[...]) >= t
    wm_ref[...] = jnp.where(m, w_ref[...], 0.0).astype(jnp.bfloat16)


def _tc_prep(weight, ps_top, t11):
    grid = 8
    return pl.pallas_call(
        _tc_prep_body,
        grid=(grid,),
        in_specs=[
            pl.BlockSpec((2048 // grid, 2048), lambda g: (g, 0)),
            pl.BlockSpec((2048 // grid, 2048), lambda g: (g, 0)),
            pl.BlockSpec(memory_space=pltpu.SMEM),
        ],
        out_specs=pl.BlockSpec((2048 // grid, 2048), lambda g: (g, 0)),
        out_shape=jax.ShapeDtypeStruct((2048, 2048), jnp.bfloat16),
    )(weight, ps_top, t11)


def _tc_matmul_body(x_ref, w_ref, b_ref, o_ref):
    x16 = x_ref[...].astype(jnp.bfloat16)
    acc = lax.dot_general(
        x16, w_ref[...], (((1,), (1,)), ((), ())),
        preferred_element_type=jnp.float32)
    o_ref[...] = acc + b_ref[...]


def _tc_matmul(x, wm, bias2d):
    bm = 512
    return pl.pallas_call(
        _tc_matmul_body,
        grid=(8192 // bm,),
        in_specs=[
            pl.BlockSpec((bm, 2048), lambda i: (i, 0)),
            pl.BlockSpec((2048, 2048), lambda i: (0, 0)),
            pl.BlockSpec((1, 2048), lambda i: (0, 0)),
        ],
        out_specs=pl.BlockSpec((bm, 2048), lambda i: (i, 0)),
        out_shape=jax.ShapeDtypeStruct((8192, 2048), jnp.float32),
        compiler_params=pltpu.CompilerParams(
            dimension_semantics=("arbitrary",)),
    )(x, wm, bias2d)


def kernel(x, weight, bias, popup_score):
    ps_flat = popup_score.reshape(-1)
    h1 = _make_sc_pass(1)(ps_flat)
    h2 = _make_sc_pass(2)(ps_flat, h1)
    h3 = _make_sc_pass(3)(ps_flat, h1, h2)
    t = _make_sc_finalize()(h1, h2, h3)
    t11 = t[:1].reshape(1, 1)
    wm = _tc_prep(weight, popup_score[:2048], t11)
    return _tc_matmul(x, wm, bias.reshape(1, 2048))


# speedup vs baseline: 147.7413x; 2.0658x over previous
import functools

import jax
import jax.numpy as jnp
from jax import lax
from jax.experimental import pallas as pl
from jax.experimental.pallas import tpu as pltpu
from jax.experimental.pallas import tpu_sc as plsc

N_TOT = 8192 * 1024
RANK_J = N_TOT // 2
NW = 32
PER_W = N_TOT // NW
CHUNK = 8192
NCHUNK = PER_W // CHUNK
NBINS = 2048
L = 16

@functools.cache
def _mesh():
    return plsc.VectorSubcoreMesh(core_axis_name="c", subcore_axis_name="s",
                                  num_cores=2, num_subcores=16)


def _worker_id():
    return lax.axis_index("s") * 2 + lax.axis_index("c")


def _find_bin(tot_ref, target):
    zeros = jnp.zeros((L,), jnp.int32)
    ones = jnp.ones((L,), jnp.int32)

    def body(c, carry):
        csum, bcnt, below = carry
        v = tot_ref[pl.ds(c * L, L)]
        cs = plsc.cumsum(v) + csum
        flags = cs < target
        bcnt = bcnt + jnp.where(flags, ones, zeros)
        below = below + jnp.where(flags, v, zeros)
        csum = csum + jnp.sum(v)
        return csum, bcnt, below

    _, bcnt, below = lax.fori_loop(0, NBINS // L, body, (zeros, zeros, zeros))
    return jnp.sum(bcnt), jnp.sum(below)


def _reduce_hist(h_ref, staging_ref, tot_ref):
    pltpu.sync_copy(h_ref, staging_ref)

    def body(c, _):
        acc = staging_ref[pl.ds(c * L, L)]
        for r in range(1, NW):
            acc = acc + staging_ref[pl.ds(r * NBINS + c * L, L)]
        tot_ref[pl.ds(c * L, L)] = acc
        return 0

    lax.fori_loop(0, NBINS // L, body, 0)


def _consume(buf_ref, hist_ref, laneoff, ones, mode, sel):
    UNROLL = 16

    def body(i, _):
        vs = [plsc.bitcast(buf_ref[pl.ds((i * UNROLL + u) * L, L)], jnp.int32)
              for u in range(UNROLL)]
        abs_ = [v & jnp.int32(0x7FFFFFFF) for v in vs]
        if mode == 1:
            idxs = [laneoff + (ab >> 20) for ab in abs_]
            for idx in idxs:
                plsc.addupdate_scatter(hist_ref, [idx], ones)
        elif mode == 2:
            preds = [(ab >> 20) == sel for ab in abs_]
            idxs = [laneoff + ((ab >> 9) & jnp.int32(0x7FF)) for ab in abs_]
            for idx, pr in zip(idxs, preds):
                plsc.addupdate_scatter(hist_ref, [idx], ones, mask=pr)
        else:
            preds = [(ab >> 9) == sel for ab in abs_]
            idxs = [laneoff + (ab & jnp.int32(0x1FF)) for ab in abs_]
            for idx, pr in zip(idxs, preds):
                plsc.addupdate_scatter(hist_ref, [idx], ones, mask=pr)
        return 0

    lax.fori_loop(0, CHUNK // (UNROLL * L), body, 0)


def _hist_body(ps_ref, out_ref, buf0, buf1, hist, totrow, staging, tot,
               sem0, sem1, mode, h1=None, h2=None):
    w = _worker_id()
    laneoff = lax.iota(jnp.int32, L) * NBINS
    ones = jnp.ones((L,), jnp.int32)

    if mode == 1:
        sel = None
    elif mode == 2:
        _reduce_hist(h1, staging, tot)
        b1, _ = _find_bin(tot, jnp.int32(RANK_J + 1))
        sel = b1
    else:
        _reduce_hist(h1, staging, tot)
        b1, below1 = _find_bin(tot, jnp.int32(RANK_J + 1))
        _reduce_hist(h2, staging, tot)
        b2, _ = _find_bin(tot, jnp.int32(RANK_J + 1) - below1)
        sel = (b1 << 11) | b2

    zeros = jnp.zeros((L,), jnp.int32)

    def zbody(c, _):
        for r in range(L):
            hist[pl.ds(r * NBINS + c * L, L)] = zeros
        return 0

    lax.fori_loop(0, NBINS // L, zbody, 0)

    base = w * PER_W

    def src(c):
        return ps_ref.at[pl.ds(base + c * CHUNK, CHUNK)]

    cp0 = pltpu.async_copy(src(0), buf0, sem0)
    cp1 = pltpu.async_copy(src(1), buf1, sem1)

    def pair(p, _):
        c = p * 2
        pltpu.make_async_copy(src(c), buf0, sem0).wait()
        _consume(buf0, hist, laneoff, ones, mode, sel)

        @pl.when(c + 2 < NCHUNK)
        def _():
            pltpu.async_copy(src(c + 2), buf0, sem0)

        pltpu.make_async_copy(src(c + 1), buf1, sem1).wait()
        _consume(buf1, hist, laneoff, ones, mode, sel)

        @pl.when(c + 3 < NCHUNK)
        def _():
            pltpu.async_copy(src(c + 3), buf1, sem1)

        return 0

    lax.fori_loop(0, NCHUNK // 2, pair, 0)
    del cp0, cp1

    def rbody(c, _):
        acc = hist[pl.ds(c * L, L)]
        for r in range(1, L):
            acc = acc + hist[pl.ds(r * NBINS + c * L, L)]
        totrow[pl.ds(c * L, L)] = acc
        return 0

    lax.fori_loop(0, NBINS // L, rbody, 0)
    pltpu.sync_copy(totrow, out_ref.at[pl.ds(w * NBINS, NBINS)])


@functools.cache
def _make_sc_pass(mode):
    scratch = [
        pltpu.VMEM((CHUNK,), jnp.float32),
        pltpu.VMEM((CHUNK,), jnp.float32),
        pltpu.VMEM((L * NBINS,), jnp.int32),
        pltpu.VMEM((NBINS,), jnp.int32),
        pltpu.VMEM((NW * NBINS,), jnp.int32),
        pltpu.VMEM((NBINS,), jnp.int32),
        pltpu.SemaphoreType.DMA,
        pltpu.SemaphoreType.DMA,
    ]
    out = jax.ShapeDtypeStruct((NW * NBINS,), jnp.int32)
    cp = pltpu.CompilerParams(needs_layout_passes=False)

    if mode == 1:
        @functools.partial(pl.kernel, out_type=out, mesh=_mesh(),
                           scratch_types=scratch, compiler_params=cp)
        def k(ps, o, b0, b1, hist, totrow, staging, tot, s0, s1):
            _hist_body(ps, o, b0, b1, hist, totrow, staging, tot, s0, s1, 1)
        return k
    if mode == 2:
        @functools.partial(pl.kernel, out_type=out, mesh=_mesh(),
                           scratch_types=scratch, compiler_params=cp)
        def k(ps, h1, o, b0, b1, hist, totrow, staging, tot, s0, s1):
            _hist_body(ps, o, b0, b1, hist, totrow, staging, tot, s0, s1, 2,
                       h1=h1)
        return k

    @functools.partial(pl.kernel, out_type=out, mesh=_mesh(),
                       scratch_types=scratch, compiler_params=cp)
    def k(ps, h1, h2, o, b0, b1, hist, totrow, staging, tot, s0, s1):
        _hist_body(ps, o, b0, b1, hist, totrow, staging, tot, s0, s1, 3,
                   h1=h1, h2=h2)
    return k


@functools.cache
def _make_sc_finalize():
    @functools.partial(pl.kernel,
                       out_type=jax.ShapeDtypeStruct((L,), jnp.float32),
                       mesh=_mesh(),
                       scratch_types=[
                           pltpu.VMEM((NW * NBINS,), jnp.int32),
                           pltpu.VMEM((NBINS,), jnp.int32),
                           pltpu.VMEM((L,), jnp.float32),
                       ],
                       compiler_params=pltpu.CompilerParams(
                           needs_layout_passes=False))
    def _sc_finalize(h1, h2, h3, out, staging, tot, tvec):
        w = _worker_id()

        @pl.when(w == 0)
        def _():
            _reduce_hist(h1, staging, tot)
            b1, below1 = _find_bin(tot, jnp.int32(RANK_J + 1))
            t2 = jnp.int32(RANK_J + 1) - below1
            _reduce_hist(h2, staging, tot)
            b2, below2 = _find_bin(tot, t2)
            t3 = t2 - below2
            _reduce_hist(h3, staging, tot)
            b3, _ = _find_bin(tot, t3)
            tbits = (b1 << 20) | (b2 << 9) | b3
            tvec[...] = plsc.bitcast(jnp.broadcast_to(tbits, (L,)),
                                     jnp.float32)
            pltpu.sync_copy(tvec, out)

    return _sc_finalize


def _tc_prep_body(w_ref, ps_ref, t_ref, wm_ref):
    t = t_ref[0, 0]
    m = jnp.abs(ps_ref[...]) >= t
    wm_ref[...] = jnp.where(m, w_ref[...], 0.0).astype(jnp.bfloat16)


def _tc_prep(weight, popup_score, t11):
    grid = 8
    return pl.pallas_call(
        _tc_prep_body,
        grid=(grid,),
        in_specs=[
            pl.BlockSpec((2048 // grid, 2048), lambda g: (g, 0)),
            pl.BlockSpec((2048 // grid, 2048), lambda g: (g, 0)),
            pl.BlockSpec(memory_space=pltpu.SMEM),
        ],
        out_specs=pl.BlockSpec((2048 // grid, 2048), lambda g: (g, 0)),
        out_shape=jax.ShapeDtypeStruct((2048, 2048), jnp.bfloat16),
    )(weight, popup_score, t11)


def _tc_matmul_body(x_ref, w_ref, b_ref, o_ref):
    x16 = x_ref[...].astype(jnp.bfloat16)
    acc = lax.dot_general(
        x16, w_ref[...], (((1,), (1,)), ((), ())),
        preferred_element_type=jnp.float32)
    o_ref[...] = acc + b_ref[...]


def _tc_matmul(x, wm, bias2d):
    bm = 512
    return pl.pallas_call(
        _tc_matmul_body,
        grid=(8192 // bm,),
        in_specs=[
            pl.BlockSpec((bm, 2048), lambda i: (i, 0)),
            pl.BlockSpec((2048, 2048), lambda i: (0, 0)),
            pl.BlockSpec((1, 2048), lambda i: (0, 0)),
        ],
        out_specs=pl.BlockSpec((bm, 2048), lambda i: (i, 0)),
        out_shape=jax.ShapeDtypeStruct((8192, 2048), jnp.float32),
        compiler_params=pltpu.CompilerParams(
            dimension_semantics=("arbitrary",)),
    )(x, wm, bias2d)


def kernel(x, weight, bias, popup_score):
    ps_flat = popup_score.reshape(-1)
    h1 = _make_sc_pass(1)(ps_flat)
    h2 = _make_sc_pass(2)(ps_flat, h1)
    h3 = _make_sc_pass(3)(ps_flat, h1, h2)
    t = _make_sc_finalize()(h1, h2, h3)
    t11 = t[:1].reshape(1, 1)
    wm = _tc_prep(weight, popup_score, t11)
    return _tc_matmul(x, wm, bias.reshape(1, 2048))


# speedup vs baseline: 155.8547x; 1.0549x over previous
import functools

import jax
import jax.numpy as jnp
from jax import lax
from jax.experimental import pallas as pl
from jax.experimental.pallas import tpu as pltpu
from jax.experimental.pallas import tpu_sc as plsc

N_TOT = 8192 * 1024
RANK_J = N_TOT // 2
NW = 32
PER_W = N_TOT // NW
ROWS = 4096
COLS = 2048
CH_ROWS = 8
CHUNK = CH_ROWS * COLS
NCHUNK = PER_W // CHUNK
RED_BATCH = 16
NBINS = 2048
L = 16

@functools.cache
def _mesh():
    return plsc.VectorSubcoreMesh(core_axis_name="c", subcore_axis_name="s",
                                  num_cores=2, num_subcores=16)


def _worker_id():
    return lax.axis_index("s") * 2 + lax.axis_index("c")


def _find_bin(tot_ref, target):
    zeros = jnp.zeros((L,), jnp.int32)
    ones = jnp.ones((L,), jnp.int32)

    def body(c, carry):
        csum, bcnt, below = carry
        v = tot_ref[pl.ds(c * L, L)]
        cs = plsc.cumsum(v) + csum
        flags = cs < target
        bcnt = bcnt + jnp.where(flags, ones, zeros)
        below = below + jnp.where(flags, v, zeros)
        csum = csum + jnp.sum(v)
        return csum, bcnt, below

    _, bcnt, below = lax.fori_loop(0, NBINS // L, body, (zeros, zeros, zeros))
    return jnp.sum(bcnt), jnp.sum(below)


def _reduce_hist(h_ref, staging_ref, tot_ref):
    for b in range(NW // RED_BATCH):
        pltpu.sync_copy(h_ref.at[pl.ds(b * RED_BATCH * NBINS,
                                       RED_BATCH * NBINS)], staging_ref)

        def body(c, _, _b=b):
            acc = staging_ref[pl.ds(c * L, L)]
            for r in range(1, RED_BATCH):
                acc = acc + staging_ref[pl.ds(r * NBINS + c * L, L)]
            if _b == 0:
                tot_ref[pl.ds(c * L, L)] = acc
            else:
                tot_ref[pl.ds(c * L, L)] = tot_ref[pl.ds(c * L, L)] + acc
            return 0

        lax.fori_loop(0, NBINS // L, body, 0)


def _consume(buf_ref, hist_ref, laneoff, ones, mode, sel):
    UNROLL = 16

    def row(r, i, _):
        vs = [plsc.bitcast(buf_ref[r, pl.ds((i * UNROLL + u) * L, L)],
                           jnp.int32)
              for u in range(UNROLL)]
        abs_ = [v & jnp.int32(0x7FFFFFFF) for v in vs]
        if mode == 1:
            idxs = [laneoff + (ab >> 20) for ab in abs_]
            for idx in idxs:
                plsc.addupdate_scatter(hist_ref, [idx], ones)
        elif mode == 2:
            preds = [(ab >> 20) == sel for ab in abs_]
            idxs = [laneoff + ((ab >> 9) & jnp.int32(0x7FF)) for ab in abs_]
            for idx, pr in zip(idxs, preds):
                plsc.addupdate_scatter(hist_ref, [idx], ones, mask=pr)
        else:
            preds = [(ab >> 9) == sel for ab in abs_]
            idxs = [laneoff + (ab & jnp.int32(0x1FF)) for ab in abs_]
            for idx, pr in zip(idxs, preds):
                plsc.addupdate_scatter(hist_ref, [idx], ones, mask=pr)
        return 0

    for r in range(CH_ROWS):
        lax.fori_loop(0, COLS // (UNROLL * L), functools.partial(row, r), 0)


def _hist_body(ps_ref, out_ref, buf0, buf1, hist, totrow, staging, tot,
               sem0, sem1, mode, h1=None, h2=None):
    w = _worker_id()
    laneoff = lax.iota(jnp.int32, L) * NBINS
    ones = jnp.ones((L,), jnp.int32)

    if mode == 1:
        sel = None
    elif mode == 2:
        _reduce_hist(h1, staging, tot)
        b1, _ = _find_bin(tot, jnp.int32(RANK_J + 1))
        sel = b1
    else:
        _reduce_hist(h1, staging, tot)
        b1, below1 = _find_bin(tot, jnp.int32(RANK_J + 1))
        _reduce_hist(h2, staging, tot)
        b2, _ = _find_bin(tot, jnp.int32(RANK_J + 1) - below1)
        sel = (b1 << 11) | b2

    zeros = jnp.zeros((L,), jnp.int32)

    def zbody(c, _):
        for r in range(L):
            hist[pl.ds(r * NBINS + c * L, L)] = zeros
        return 0

    lax.fori_loop(0, NBINS // L, zbody, 0)

    base = w * (ROWS // NW)

    def src(c):
        return ps_ref.at[pl.ds(base + c * CH_ROWS, CH_ROWS)]

    cp0 = pltpu.async_copy(src(0), buf0, sem0)
    cp1 = pltpu.async_copy(src(1), buf1, sem1)

    def pair(p, _):
        c = p * 2
        pltpu.make_async_copy(src(c), buf0, sem0).wait()
        _consume(buf0, hist, laneoff, ones, mode, sel)

        @pl.when(c + 2 < NCHUNK)
        def _():
            pltpu.async_copy(src(c + 2), buf0, sem0)

        pltpu.make_async_copy(src(c + 1), buf1, sem1).wait()
        _consume(buf1, hist, laneoff, ones, mode, sel)

        @pl.when(c + 3 < NCHUNK)
        def _():
            pltpu.async_copy(src(c + 3), buf1, sem1)

        return 0

    lax.fori_loop(0, NCHUNK // 2, pair, 0)
    del cp0, cp1

    def rbody(c, _):
        acc = hist[pl.ds(c * L, L)]
        for r in range(1, L):
            acc = acc + hist[pl.ds(r * NBINS + c * L, L)]
        totrow[pl.ds(c * L, L)] = acc
        return 0

    lax.fori_loop(0, NBINS // L, rbody, 0)
    pltpu.sync_copy(totrow, out_ref.at[pl.ds(w * NBINS, NBINS)])


@functools.cache
def _make_sc_pass(mode):
    scratch = [
        pltpu.VMEM((CH_ROWS, COLS), jnp.float32),
        pltpu.VMEM((CH_ROWS, COLS), jnp.float32),
        pltpu.VMEM((L * NBINS,), jnp.int32),
        pltpu.VMEM((NBINS,), jnp.int32),
        pltpu.VMEM((RED_BATCH * NBINS,), jnp.int32),
        pltpu.VMEM((NBINS,), jnp.int32),
        pltpu.SemaphoreType.DMA,
        pltpu.SemaphoreType.DMA,
    ]
    out = jax.ShapeDtypeStruct((NW * NBINS,), jnp.int32)
    cp = pltpu.CompilerParams(needs_layout_passes=False)

    if mode == 1:
        @functools.partial(pl.kernel, out_type=out, mesh=_mesh(),
                           scratch_types=scratch, compiler_params=cp)
        def k(ps, o, b0, b1, hist, totrow, staging, tot, s0, s1):
            _hist_body(ps, o, b0, b1, hist, totrow, staging, tot, s0, s1, 1)
        return k
    if mode == 2:
        @functools.partial(pl.kernel, out_type=out, mesh=_mesh(),
                           scratch_types=scratch, compiler_params=cp)
        def k(ps, h1, o, b0, b1, hist, totrow, staging, tot, s0, s1):
            _hist_body(ps, o, b0, b1, hist, totrow, staging, tot, s0, s1, 2,
                       h1=h1)
        return k

    @functools.partial(pl.kernel, out_type=out, mesh=_mesh(),
                       scratch_types=scratch, compiler_params=cp)
    def k(ps, h1, h2, o, b0, b1, hist, totrow, staging, tot, s0, s1):
        _hist_body(ps, o, b0, b1, hist, totrow, staging, tot, s0, s1, 3,
                   h1=h1, h2=h2)
    return k


@functools.cache
def _make_sc_finalize():
    @functools.partial(pl.kernel,
                       out_type=jax.ShapeDtypeStruct((L,), jnp.float32),
                       mesh=_mesh(),
                       scratch_types=[
                           pltpu.VMEM((RED_BATCH * NBINS,), jnp.int32),
                           pltpu.VMEM((NBINS,), jnp.int32),
                           pltpu.VMEM((L,), jnp.float32),
                       ],
                       compiler_params=pltpu.CompilerParams(
                           needs_layout_passes=False))
    def _sc_finalize(h1, h2, h3, out, staging, tot, tvec):
        w = _worker_id()

        @pl.when(w == 0)
        def _():
            _reduce_hist(h1, staging, tot)
            b1, below1 = _find_bin(tot, jnp.int32(RANK_J + 1))
            t2 = jnp.int32(RANK_J + 1) - below1
            _reduce_hist(h2, staging, tot)
            b2, below2 = _find_bin(tot, t2)
            t3 = t2 - below2
            _reduce_hist(h3, staging, tot)
            b3, _ = _find_bin(tot, t3)
            tbits = (b1 << 20) | (b2 << 9) | b3
            tvec[...] = plsc.bitcast(jnp.broadcast_to(tbits, (L,)),
                                     jnp.float32)
            pltpu.sync_copy(tvec, out)

    return _sc_finalize


def _tc_prep_body(w_ref, ps_ref, t_ref, wm_ref):
    t = t_ref[0, 0]
    m = jnp.abs(ps_ref[...]) >= t
    wm_ref[...] = jnp.where(m, w_ref[...], 0.0).astype(jnp.bfloat16)


def _tc_prep(weight, popup_score, t11):
    grid = 8
    return pl.pallas_call(
        _tc_prep_body,
        grid=(grid,),
        in_specs=[
            pl.BlockSpec((2048 // grid, 2048), lambda g: (g, 0)),
            pl.BlockSpec((2048 // grid, 2048), lambda g: (g, 0)),
            pl.BlockSpec(memory_space=pltpu.SMEM),
        ],
        out_specs=pl.BlockSpec((2048 // grid, 2048), lambda g: (g, 0)),
        out_shape=jax.ShapeDtypeStruct((2048, 2048), jnp.bfloat16),
    )(weight, popup_score, t11)


def _tc_matmul_body(x_ref, w_ref, b_ref, o_ref):
    x16 = x_ref[...].astype(jnp.bfloat16)
    acc = lax.dot_general(
        x16, w_ref[...], (((1,), (1,)), ((), ())),
        preferred_element_type=jnp.float32)
    o_ref[...] = acc + b_ref[...]


def _tc_matmul(x, wm, bias2d):
    bm = 512
    return pl.pallas_call(
        _tc_matmul_body,
        grid=(8192 // bm,),
        in_specs=[
            pl.BlockSpec((bm, 2048), lambda i: (i, 0)),
            pl.BlockSpec((2048, 2048), lambda i: (0, 0)),
            pl.BlockSpec((1, 2048), lambda i: (0, 0)),
        ],
        out_specs=pl.BlockSpec((bm, 2048), lambda i: (i, 0)),
        out_shape=jax.ShapeDtypeStruct((8192, 2048), jnp.float32),
        compiler_params=pltpu.CompilerParams(
            dimension_semantics=("arbitrary",)),
    )(x, wm, bias2d)


def kernel(x, weight, bias, popup_score):
    h1 = _make_sc_pass(1)(popup_score)
    h2 = _make_sc_pass(2)(popup_score, h1)
    h3 = _make_sc_pass(3)(popup_score, h1, h2)
    t = _make_sc_finalize()(h1, h2, h3)
    t11 = t[:1].reshape(1, 1)
    wm = _tc_prep(weight, popup_score, t11)
    return _tc_matmul(x, wm, bias.reshape(1, 2048))


# speedup vs baseline: 172.2067x; 1.1049x over previous
import functools

import jax
import jax.numpy as jnp
from jax import lax
from jax.experimental import pallas as pl
from jax.experimental.pallas import tpu as pltpu
from jax.experimental.pallas import tpu_sc as plsc

N_TOT = 8192 * 1024
RANK_J = N_TOT // 2
NW = 32
PER_W = N_TOT // NW
ROWS = 4096
COLS = 2048
CH_ROWS = 8
CHUNK = CH_ROWS * COLS
NCHUNK = PER_W // CHUNK
RED_BATCH = 16
NBINS = 2048
L = 16

@functools.cache
def _mesh():
    return plsc.VectorSubcoreMesh(core_axis_name="c", subcore_axis_name="s",
                                  num_cores=2, num_subcores=16)


def _worker_id():
    return lax.axis_index("s") * 2 + lax.axis_index("c")


def _find_bin(tot_ref, target):
    zeros = jnp.zeros((L,), jnp.int32)
    ones = jnp.ones((L,), jnp.int32)

    def body(c, carry):
        csum, bcnt, below = carry
        v = tot_ref[pl.ds(c * L, L)]
        cs = plsc.cumsum(v) + csum
        flags = cs < target
        bcnt = bcnt + jnp.where(flags, ones, zeros)
        below = below + jnp.where(flags, v, zeros)
        csum = csum + jnp.sum(v)
        return csum, bcnt, below

    _, bcnt, below = lax.fori_loop(0, NBINS // L, body, (zeros, zeros, zeros))
    return jnp.sum(bcnt), jnp.sum(below)


def _reduce_hist(h_ref, staging_ref, tot_ref):
    for b in range(NW // RED_BATCH):
        pltpu.sync_copy(h_ref.at[pl.ds(b * RED_BATCH, RED_BATCH)],
                        staging_ref)

        def body(c, _, _b=b):
            acc = staging_ref[0, pl.ds(c * L, L)]
            for r in range(1, RED_BATCH):
                acc = acc + staging_ref[r, pl.ds(c * L, L)]
            if _b == 0:
                tot_ref[pl.ds(c * L, L)] = acc
            else:
                tot_ref[pl.ds(c * L, L)] = tot_ref[pl.ds(c * L, L)] + acc
            return 0

        lax.fori_loop(0, NBINS // L, body, 0)


def _consume(buf_ref, hist_ref, laneoff, ones, mode, sel):
    UNROLL = 16

    def row(r, i, _):
        vs = [plsc.bitcast(buf_ref[r, pl.ds((i * UNROLL + u) * L, L)],
                           jnp.int32)
              for u in range(UNROLL)]
        abs_ = [v & jnp.int32(0x7FFFFFFF) for v in vs]
        if mode == 1:
            idxs = [laneoff + (ab >> 20) for ab in abs_]
            for idx in idxs:
                plsc.addupdate_scatter(hist_ref, [idx], ones)
        elif mode == 2:
            preds = [(ab >> 20) == sel for ab in abs_]
            idxs = [laneoff + ((ab >> 9) & jnp.int32(0x7FF)) for ab in abs_]
            for idx, pr in zip(idxs, preds):
                plsc.addupdate_scatter(hist_ref, [idx], ones, mask=pr)
        else:
            preds = [(ab >> 9) == sel for ab in abs_]
            idxs = [laneoff + (ab & jnp.int32(0x1FF)) for ab in abs_]
            for idx, pr in zip(idxs, preds):
                plsc.addupdate_scatter(hist_ref, [idx], ones, mask=pr)
        return 0

    for r in range(CH_ROWS):
        lax.fori_loop(0, COLS // (UNROLL * L), functools.partial(row, r), 0)


def _hist_body(ps_ref, out_ref, buf0, buf1, hist, totrow, staging, tot,
               sem0, sem1, mode, h1=None, h2=None):
    w = _worker_id()
    laneoff = lax.iota(jnp.int32, L) * NBINS
    ones = jnp.ones((L,), jnp.int32)

    if mode == 1:
        sel = None
    elif mode == 2:
        _reduce_hist(h1, staging, tot)
        b1, _ = _find_bin(tot, jnp.int32(RANK_J + 1))
        sel = b1
    else:
        _reduce_hist(h1, staging, tot)
        b1, below1 = _find_bin(tot, jnp.int32(RANK_J + 1))
        _reduce_hist(h2, staging, tot)
        b2, _ = _find_bin(tot, jnp.int32(RANK_J + 1) - below1)
        sel = (b1 << 11) | b2

    zeros = jnp.zeros((L,), jnp.int32)

    def zbody(c, _):
        for r in range(L):
            hist[pl.ds(r * NBINS + c * L, L)] = zeros
        return 0

    lax.fori_loop(0, NBINS // L, zbody, 0)

    base = w * (ROWS // NW)

    def src(c):
        return ps_ref.at[pl.ds(base + c * CH_ROWS, CH_ROWS)]

    cp0 = pltpu.async_copy(src(0), buf0, sem0)
    cp1 = pltpu.async_copy(src(1), buf1, sem1)

    def pair(p, _):
        c = p * 2
        pltpu.make_async_copy(src(c), buf0, sem0).wait()
        _consume(buf0, hist, laneoff, ones, mode, sel)

        @pl.when(c + 2 < NCHUNK)
        def _():
            pltpu.async_copy(src(c + 2), buf0, sem0)

        pltpu.make_async_copy(src(c + 1), buf1, sem1).wait()
        _consume(buf1, hist, laneoff, ones, mode, sel)

        @pl.when(c + 3 < NCHUNK)
        def _():
            pltpu.async_copy(src(c + 3), buf1, sem1)

        return 0

    lax.fori_loop(0, NCHUNK // 2, pair, 0)
    del cp0, cp1

    def rbody(c, _):
        acc = hist[pl.ds(c * L, L)]
        for r in range(1, L):
            acc = acc + hist[pl.ds(r * NBINS + c * L, L)]
        totrow[0, pl.ds(c * L, L)] = acc
        return 0

    lax.fori_loop(0, NBINS // L, rbody, 0)
    pltpu.sync_copy(totrow, out_ref.at[pl.ds(w, 1)])


@functools.cache
def _make_sc_pass(mode):
    scratch = [
        pltpu.VMEM((CH_ROWS, COLS), jnp.float32),
        pltpu.VMEM((CH_ROWS, COLS), jnp.float32),
        pltpu.VMEM((L * NBINS,), jnp.int32),
        pltpu.VMEM((1, NBINS), jnp.int32),
        pltpu.VMEM((RED_BATCH, NBINS), jnp.int32),
        pltpu.VMEM((NBINS,), jnp.int32),
        pltpu.SemaphoreType.DMA,
        pltpu.SemaphoreType.DMA,
    ]
    out = jax.ShapeDtypeStruct((NW, NBINS), jnp.int32)
    cp = pltpu.CompilerParams(needs_layout_passes=False)

    if mode == 1:
        @functools.partial(pl.kernel, out_type=out, mesh=_mesh(),
                           scratch_types=scratch, compiler_params=cp)
        def k(ps, o, b0, b1, hist, totrow, staging, tot, s0, s1):
            _hist_body(ps, o, b0, b1, hist, totrow, staging, tot, s0, s1, 1)
        return k
    if mode == 2:
        @functools.partial(pl.kernel, out_type=out, mesh=_mesh(),
                           scratch_types=scratch, compiler_params=cp)
        def k(ps, h1, o, b0, b1, hist, totrow, staging, tot, s0, s1):
            _hist_body(ps, o, b0, b1, hist, totrow, staging, tot, s0, s1, 2,
                       h1=h1)
        return k

    @functools.partial(pl.kernel, out_type=out, mesh=_mesh(),
                       scratch_types=scratch, compiler_params=cp)
    def k(ps, h1, h2, o, b0, b1, hist, totrow, staging, tot, s0, s1):
        _hist_body(ps, o, b0, b1, hist, totrow, staging, tot, s0, s1, 3,
                   h1=h1, h2=h2)
    return k


def _cumsum_lanes(v):
    k = 1
    while k < NBINS:
        shifted = jnp.concatenate(
            [jnp.zeros((1, k), v.dtype), v[:, :NBINS - k]], axis=1)
        v = v + shifted
        k *= 2
    return v


def _tc_find(tot, target):
    cs = _cumsum_lanes(tot)
    flags = cs < target
    B = jnp.sum(flags.astype(jnp.int32))
    below = jnp.sum(jnp.where(flags, tot, jnp.zeros_like(tot)))
    return B, below


def _tc_prep_body(w_ref, ps_ref, h1_ref, h2_ref, h3_ref, wm_ref, ts_ref):
    @pl.when(pl.program_id(0) == 0)
    def _():
        t1 = jnp.int32(RANK_J + 1)
        b1, below1 = _tc_find(jnp.sum(h1_ref[...], axis=0, keepdims=True), t1)
        t2 = t1 - below1
        b2, below2 = _tc_find(jnp.sum(h2_ref[...], axis=0, keepdims=True), t2)
        t3 = t2 - below2
        b3, _ = _tc_find(jnp.sum(h3_ref[...], axis=0, keepdims=True), t3)
        ts_ref[0] = (b1 << 20) | (b2 << 9) | b3

    tb = ts_ref[0]
    psb = lax.bitcast_convert_type(ps_ref[...], jnp.int32)
    psb = psb & jnp.int32(0x7FFFFFFF)
    m = psb >= tb
    wm_ref[...] = jnp.where(m, w_ref[...], 0.0).astype(jnp.bfloat16)


def _tc_prep(weight, popup_score, h1, h2, h3):
    grid = 8
    return pl.pallas_call(
        _tc_prep_body,
        grid=(grid,),
        in_specs=[
            pl.BlockSpec((2048 // grid, 2048), lambda g: (g, 0)),
            pl.BlockSpec((2048 // grid, 2048), lambda g: (g, 0)),
            pl.BlockSpec((NW, NBINS), lambda g: (0, 0)),
            pl.BlockSpec((NW, NBINS), lambda g: (0, 0)),
            pl.BlockSpec((NW, NBINS), lambda g: (0, 0)),
        ],
        out_specs=pl.BlockSpec((2048 // grid, 2048), lambda g: (g, 0)),
        out_shape=jax.ShapeDtypeStruct((2048, 2048), jnp.bfloat16),
        scratch_shapes=[pltpu.SMEM((1,), jnp.int32)],
    )(weight, popup_score, h1, h2, h3)


def _tc_matmul_body(x_ref, w_ref, b_ref, o_ref):
    x16 = x_ref[...].astype(jnp.bfloat16)
    acc = lax.dot_general(
        x16, w_ref[...], (((1,), (1,)), ((), ())),
        preferred_element_type=jnp.float32)
    o_ref[...] = acc + b_ref[...]


def _tc_matmul(x, wm, bias2d):
    bm = 512
    return pl.pallas_call(
        _tc_matmul_body,
        grid=(8192 // bm,),
        in_specs=[
            pl.BlockSpec((bm, 2048), lambda i: (i, 0)),
            pl.BlockSpec((2048, 2048), lambda i: (0, 0)),
            pl.BlockSpec((1, 2048), lambda i: (0, 0)),
        ],
        out_specs=pl.BlockSpec((bm, 2048), lambda i: (i, 0)),
        out_shape=jax.ShapeDtypeStruct((8192, 2048), jnp.float32),
        compiler_params=pltpu.CompilerParams(
            dimension_semantics=("arbitrary",)),
    )(x, wm, bias2d)


def kernel(x, weight, bias, popup_score):
    h1 = _make_sc_pass(1)(popup_score)
    h2 = _make_sc_pass(2)(popup_score, h1)
    h3 = _make_sc_pass(3)(popup_score, h1, h2)
    wm = _tc_prep(weight, popup_score, h1, h2, h3)
    return _tc_matmul(x, wm, bias.reshape(1, 2048))
